# baseline (speedup 1.0000x reference)
import functools
import jax
import jax.numpy as jnp
from jax import lax
from jax.experimental import pallas as pl
from jax.experimental.pallas import tpu as pltpu
from jax.experimental.pallas import tpu_sc as plsc

BATCH = 16384
SEQ = 5
DIM = 64
NROWS = 1000000
LANES = 16
NC, NS = 2, 16
NW = NC * NS
BPW = BATCH // NW
G = 128
KG = BPW // G
BLK = 32768


def _proj_body(w3_ref, tt_ref, *out_refs):
    wm = w3_ref[0] - w3_ref[1]
    res = jax.lax.dot_general(
        wm, tt_ref[...], (((1,), (0,)), ((), ())),
        preferred_element_type=jnp.float32)
    for l, o in enumerate(out_refs):
        o[...] = res[l]


@jax.jit
def _proj(w3, tt):
    grid = (NROWS + BLK - 1) // BLK
    return pl.pallas_call(
        _proj_body,
        grid=(grid,),
        in_specs=[
            pl.BlockSpec((2, SEQ, DIM), lambda i: (0, 0, 0)),
            pl.BlockSpec((DIM, BLK), lambda i: (0, i)),
        ],
        out_specs=[pl.BlockSpec((BLK,), lambda i: (i,)) for _ in range(SEQ)],
        out_shape=[jax.ShapeDtypeStruct((NROWS,), jnp.float32)
                   for _ in range(SEQ)],
    )(w3, tt)


def _sc_body(x3_hbm, p0_hbm, p1_hbm, p2_hbm, p3_hbm, p4_hbm, b_hbm,
             out_hbm, idx_v, g_v, db_v, out_v, sem):
    wid = lax.axis_index("s") * NC + lax.axis_index("c")
    proj = (p0_hbm, p1_hbm, p2_hbm, p3_hbm, p4_hbm)

    pltpu.sync_copy(b_hbm, db_v)
    for l in range(SEQ):
        pltpu.sync_copy(x3_hbm.at[l, pl.ds(wid * KG, KG)], idx_v.at[l])

    copies = []
    for l in range(SEQ):
        for k in range(KG):
            copies.append(pltpu.async_copy(
                proj[l].at[idx_v.at[l, k]], g_v.at[l, k], sem))
    for c in copies:
        c.wait()

    db = db_v[...]
    ones = jnp.zeros((LANES,), jnp.float32) + 1.0
    for k in range(KG):
        for ig in range(G // LANES):
            sl = pl.ds(ig * LANES, LANES)
            delta = g_v[0, k, sl] + g_v[1, k, sl] + g_v[2, k, sl] \
                + g_v[3, k, sl] + g_v[4, k, sl] + db
            p0 = ones / (ones + jnp.exp(-delta))
            off = k * G + ig * LANES
            out_v[0, pl.ds(off, LANES)] = p0
            out_v[1, pl.ds(off, LANES)] = ones - p0

    base = wid * BPW
    pltpu.sync_copy(out_v.at[0], out_hbm.at[0, pl.ds(base, BPW)])
    pltpu.sync_copy(out_v.at[1], out_hbm.at[1, pl.ds(base, BPW)])


@jax.jit
def _run(x3, p0, p1, p2, p3, p4, b):
    mesh = plsc.VectorSubcoreMesh(core_axis_name="c", subcore_axis_name="s")
    f = pl.kernel(
        _sc_body,
        out_type=jax.ShapeDtypeStruct((2, BATCH), jnp.float32),
        mesh=mesh,
        scratch_types=[
            pltpu.VMEM((SEQ, KG, G), jnp.int32),
            pltpu.VMEM((SEQ, KG, G), jnp.float32),
            pltpu.VMEM((LANES,), jnp.float32),
            pltpu.VMEM((2, BPW), jnp.float32),
            pltpu.SemaphoreType.DMA,
        ],
        compiler_params=pltpu.CompilerParams(
            needs_layout_passes=False, use_tc_tiling_on_sc=False),
    )
    return f(x3, p0, p1, p2, p3, p4, b)


def kernel(x, table, W, b):
    projs = _proj(W.reshape(2, SEQ, DIM), table.T)
    x3 = x.astype(jnp.int32).T.reshape(SEQ, BATCH // G, G)
    params = jnp.full((LANES,), b[0] - b[1], jnp.float32)
    out2 = _run(x3, *projs, params)
    return out2.T


# speedup vs baseline: 1.0191x; 1.0191x over previous
import functools
import jax
import jax.numpy as jnp
from jax import lax
from jax.experimental import pallas as pl
from jax.experimental.pallas import tpu as pltpu
from jax.experimental.pallas import tpu_sc as plsc

BATCH = 16384
SEQ = 5
DIM = 64
NROWS = 1000000
LANES = 16
NC, NS = 2, 16
NW = NC * NS
BPW = BATCH // NW
G = 128
KG = BPW // G
BLK = 32768


def _proj_body(w3_ref, tt_ref, *out_refs):
    wm = w3_ref[0] - w3_ref[1]
    res = jax.lax.dot_general(
        wm, tt_ref[...], (((1,), (0,)), ((), ())),
        preferred_element_type=jnp.float32)
    for l, o in enumerate(out_refs):
        o[...] = res[l]


@jax.jit
def _proj(w3, tt):
    grid = (NROWS + BLK - 1) // BLK
    return pl.pallas_call(
        _proj_body,
        grid=(grid,),
        in_specs=[
            pl.BlockSpec((2, SEQ, DIM), lambda i: (0, 0, 0)),
            pl.BlockSpec((DIM, BLK), lambda i: (0, i)),
        ],
        out_specs=[pl.BlockSpec((BLK,), lambda i: (i,)) for _ in range(SEQ)],
        out_shape=[jax.ShapeDtypeStruct((NROWS,), jnp.float32)
                   for _ in range(SEQ)],
    )(w3, tt)


def _sc_body(x3_hbm, p0_hbm, p1_hbm, p2_hbm, p3_hbm, p4_hbm, b_hbm,
             out_hbm, idx_v, g_v, db_v, out_v, sem):
    wid = lax.axis_index("s") * NC + lax.axis_index("c")
    proj = (p0_hbm, p1_hbm, p2_hbm, p3_hbm, p4_hbm)

    pltpu.sync_copy(b_hbm, db_v)
    for l in range(SEQ):
        pltpu.sync_copy(x3_hbm.at[l, pl.ds(wid * KG, KG)], idx_v.at[l])

    copies = []
    for l in range(SEQ):
        for k in range(KG):
            copies.append(pltpu.async_copy(
                proj[l].at[idx_v.at[l, k]], g_v.at[l, k], sem))
    for c in copies:
        c.wait()

    db = db_v[...]
    ones = jnp.zeros((LANES,), jnp.float32) + 1.0
    for k in range(KG):
        for ig in range(G // LANES):
            sl = pl.ds(ig * LANES, LANES)
            delta = g_v[0, k, sl] + g_v[1, k, sl] + g_v[2, k, sl] \
                + g_v[3, k, sl] + g_v[4, k, sl] + db
            p0 = ones / (ones + jnp.exp(-delta))
            out_v[k, 0, sl] = p0
            out_v[k, 1, sl] = ones - p0

    pltpu.sync_copy(out_v, out_hbm.at[pl.ds(wid * KG, KG)])


@jax.jit
def _run(x3, p0, p1, p2, p3, p4, b):
    mesh = plsc.VectorSubcoreMesh(core_axis_name="c", subcore_axis_name="s")
    f = pl.kernel(
        _sc_body,
        out_type=jax.ShapeDtypeStruct((BATCH // G, 2, G), jnp.float32),
        mesh=mesh,
        scratch_types=[
            pltpu.VMEM((SEQ, KG, G), jnp.int32),
            pltpu.VMEM((SEQ, KG, G), jnp.float32),
            pltpu.VMEM((LANES,), jnp.float32),
            pltpu.VMEM((KG, 2, G), jnp.float32),
            pltpu.SemaphoreType.DMA,
        ],
        compiler_params=pltpu.CompilerParams(
            needs_layout_passes=False, use_tc_tiling_on_sc=False),
    )
    return f(x3, p0, p1, p2, p3, p4, b)


def kernel(x, table, W, b):
    projs = _proj(W.reshape(2, SEQ, DIM), table.T)
    x3 = x.astype(jnp.int32).T.reshape(SEQ, BATCH // G, G)
    params = jnp.full((LANES,), b[0] - b[1], jnp.float32)
    out3 = _run(x3, *projs, params)
    return out3.transpose(0, 2, 1).reshape(BATCH, 2)


# speedup vs baseline: 1.0318x; 1.0125x over previous
import functools
import jax
import jax.numpy as jnp
from jax import lax
from jax.experimental import pallas as pl
from jax.experimental.pallas import tpu as pltpu
from jax.experimental.pallas import tpu_sc as plsc

BATCH = 16384
SEQ = 5
DIM = 64
NROWS = 1000000
LANES = 16
NC, NS = 2, 16
NW = NC * NS
BPW = BATCH // NW
G = 128
KG = BPW // G
BLK = 32768


def _proj_body(w3_ref, xt_ref, tt_ref, x3_ref, *out_refs):
    @pl.when(pl.program_id(0) == 0)
    def _():
        x3_ref[...] = xt_ref[...].reshape(SEQ, BATCH // G, G)

    wm = w3_ref[0] - w3_ref[1]
    res = jax.lax.dot_general(
        wm, tt_ref[...], (((1,), (0,)), ((), ())),
        preferred_element_type=jnp.float32)
    for l, o in enumerate(out_refs):
        o[...] = res[l]


@jax.jit
def _proj(w3, xt, tt):
    grid = (NROWS + BLK - 1) // BLK
    return pl.pallas_call(
        _proj_body,
        grid=(grid,),
        in_specs=[
            pl.BlockSpec((2, SEQ, DIM), lambda i: (0, 0, 0)),
            pl.BlockSpec((SEQ, BATCH), lambda i: (0, 0)),
            pl.BlockSpec((DIM, BLK), lambda i: (0, i)),
        ],
        out_specs=[pl.BlockSpec((SEQ, BATCH // G, G), lambda i: (0, 0, 0))]
        + [pl.BlockSpec((BLK,), lambda i: (i,)) for _ in range(SEQ)],
        out_shape=[jax.ShapeDtypeStruct((SEQ, BATCH // G, G), jnp.int32)]
        + [jax.ShapeDtypeStruct((NROWS,), jnp.float32) for _ in range(SEQ)],
    )(w3, xt, tt)


def _sc_body(x3_hbm, p0_hbm, p1_hbm, p2_hbm, p3_hbm, p4_hbm, b_hbm,
             out_hbm, idx_v, g_v, db_v, out_v, sem):
    wid = lax.axis_index("s") * NC + lax.axis_index("c")
    proj = (p0_hbm, p1_hbm, p2_hbm, p3_hbm, p4_hbm)

    pltpu.sync_copy(b_hbm, db_v)
    for l in range(SEQ):
        pltpu.sync_copy(x3_hbm.at[l, pl.ds(wid * KG, KG)], idx_v.at[l])

    copies = []
    for l in range(SEQ):
        for k in range(KG):
            copies.append(pltpu.async_copy(
                proj[l].at[idx_v.at[l, k]], g_v.at[l, k], sem))
    for c in copies:
        c.wait()

    db = db_v[...]
    ones = jnp.zeros((LANES,), jnp.float32) + 1.0
    for k in range(KG):
        for ig in range(G // LANES):
            sl = pl.ds(ig * LANES, LANES)
            delta = g_v[0, k, sl] + g_v[1, k, sl] + g_v[2, k, sl] \
                + g_v[3, k, sl] + g_v[4, k, sl] + db
            p0 = ones / (ones + jnp.exp(-delta))
            out_v[k, 0, sl] = p0
            out_v[k, 1, sl] = ones - p0

    pltpu.sync_copy(out_v, out_hbm.at[pl.ds(wid * KG, KG)])


@jax.jit
def _run(x3, p0, p1, p2, p3, p4, b):
    mesh = plsc.VectorSubcoreMesh(core_axis_name="c", subcore_axis_name="s")
    f = pl.kernel(
        _sc_body,
        out_type=jax.ShapeDtypeStruct((BATCH // G, 2, G), jnp.float32),
        mesh=mesh,
        scratch_types=[
            pltpu.VMEM((SEQ, KG, G), jnp.int32),
            pltpu.VMEM((SEQ, KG, G), jnp.float32),
            pltpu.VMEM((LANES,), jnp.float32),
            pltpu.VMEM((KG, 2, G), jnp.float32),
            pltpu.SemaphoreType.DMA,
        ],
        compiler_params=pltpu.CompilerParams(
            needs_layout_passes=False, use_tc_tiling_on_sc=False),
    )
    return f(x3, p0, p1, p2, p3, p4, b)


def kernel(x, table, W, b):
    x3, *projs = _proj(W.reshape(2, SEQ, DIM), x.astype(jnp.int32).T, table.T)
    params = jnp.full((LANES,), b[0] - b[1], jnp.float32)
    out3 = _run(x3, *projs, params)
    return out3.transpose(0, 2, 1).reshape(BATCH, 2)


# speedup vs baseline: 1.0551x; 1.0226x over previous
import functools
import jax
import jax.numpy as jnp
from jax import lax
from jax.experimental import pallas as pl
from jax.experimental.pallas import tpu as pltpu
from jax.experimental.pallas import tpu_sc as plsc

BATCH = 16384
SEQ = 5
DIM = 64
NROWS = 1000000
LANES = 16
NC, NS = 2, 16
NW = NC * NS
BPW = BATCH // NW
G = 128
KG = BPW // G
BLK = 32768


def _proj_body(w3_ref, xt_ref, tt_ref, x3_ref, *out_refs):
    @pl.when(pl.program_id(0) == 0)
    def _():
        x3_ref[...] = xt_ref[...].reshape(SEQ, BATCH // G, G)

    wm = w3_ref[0] - w3_ref[1]
    res = jax.lax.dot_general(
        wm, tt_ref[...], (((1,), (0,)), ((), ())),
        preferred_element_type=jnp.float32)
    for l, o in enumerate(out_refs):
        o[...] = res[l]


@jax.jit
def _proj(w3, xt, tt):
    grid = (NROWS + BLK - 1) // BLK
    return pl.pallas_call(
        _proj_body,
        grid=(grid,),
        in_specs=[
            pl.BlockSpec((2, SEQ, DIM), lambda i: (0, 0, 0)),
            pl.BlockSpec((SEQ, BATCH), lambda i: (0, 0)),
            pl.BlockSpec((DIM, BLK), lambda i: (0, i)),
        ],
        out_specs=[pl.BlockSpec((SEQ, BATCH // G, G), lambda i: (0, 0, 0))]
        + [pl.BlockSpec((BLK,), lambda i: (i,)) for _ in range(SEQ)],
        out_shape=[jax.ShapeDtypeStruct((SEQ, BATCH // G, G), jnp.int32)]
        + [jax.ShapeDtypeStruct((NROWS,), jnp.float32) for _ in range(SEQ)],
    )(w3, xt, tt)


def _sc_body(x3_hbm, p0_hbm, p1_hbm, p2_hbm, p3_hbm, p4_hbm, b_hbm,
             out_hbm, idx_v, g_v, db_v, out_v, sem):
    wid = lax.axis_index("s") * NC + lax.axis_index("c")
    proj = (p0_hbm, p1_hbm, p2_hbm, p3_hbm, p4_hbm)

    stage = [pltpu.async_copy(b_hbm, db_v, sem)]
    for l in range(SEQ):
        stage.append(pltpu.async_copy(
            x3_hbm.at[l, pl.ds(wid * KG, KG)], idx_v.at[l], sem))
    copies = []
    for l in range(SEQ):
        if l == 0:
            for c in stage:
                c.wait()
        for k in range(KG):
            copies.append(pltpu.async_copy(
                proj[l].at[idx_v.at[l, k]], g_v.at[l, k], sem))
    for c in copies:
        c.wait()

    db = db_v[...]
    ones = jnp.zeros((LANES,), jnp.float32) + 1.0
    for k in range(KG):
        for ig in range(G // LANES):
            sl = pl.ds(ig * LANES, LANES)
            delta = g_v[0, k, sl] + g_v[1, k, sl] + g_v[2, k, sl] \
                + g_v[3, k, sl] + g_v[4, k, sl] + db
            p0 = ones / (ones + jnp.exp(-delta))
            out_v[k, 0, sl] = p0
            out_v[k, 1, sl] = ones - p0

    pltpu.sync_copy(out_v, out_hbm.at[pl.ds(wid * KG, KG)])


@jax.jit
def _run(x3, p0, p1, p2, p3, p4, b):
    mesh = plsc.VectorSubcoreMesh(core_axis_name="c", subcore_axis_name="s")
    f = pl.kernel(
        _sc_body,
        out_type=jax.ShapeDtypeStruct((BATCH // G, 2, G), jnp.float32),
        mesh=mesh,
        scratch_types=[
            pltpu.VMEM((SEQ, KG, G), jnp.int32),
            pltpu.VMEM((SEQ, KG, G), jnp.float32),
            pltpu.VMEM((LANES,), jnp.float32),
            pltpu.VMEM((KG, 2, G), jnp.float32),
            pltpu.SemaphoreType.DMA,
        ],
        compiler_params=pltpu.CompilerParams(
            needs_layout_passes=False, use_tc_tiling_on_sc=False),
    )
    return f(x3, p0, p1, p2, p3, p4, b)


def kernel(x, table, W, b):
    x3, *projs = _proj(W.reshape(2, SEQ, DIM), x.astype(jnp.int32).T, table.T)
    params = jnp.full((LANES,), b[0] - b[1], jnp.float32)
    out3 = _run(x3, *projs, params)
    return out3.transpose(0, 2, 1).reshape(BATCH, 2)
